# baseline (speedup 1.0000x reference)
import functools

import jax
import jax.numpy as jnp
from jax import lax
from jax.experimental import pallas as pl
from jax.experimental.pallas import tpu as pltpu
from jax.experimental.pallas import tpu_sc as plsc

M_ROWS = 1000000
DIM = 16
LANES = 16
NUM_CORES = 2
NUM_SUBCORES = 16
NUM_WORKERS = NUM_CORES * NUM_SUBCORES

CHUNK_COLS = 3072
GROUPS = CHUNK_COLS // LANES
CHUNKS_PER_WORKER = 4
SC_ROWS = CHUNK_COLS * NUM_WORKERS * CHUNKS_PER_WORKER
TC_BLOCK = 65536
TC_ROWS = M_ROWS - SC_ROWS


def _sc_body(itT_hbm, ubc_hbm, out_hbm,
             ub_v, bufA0, bufB0, bufA1, bufB1, out0, out1,
             insem0, insem1, outsem0, outsem1):
    wid = lax.axis_index("s") * NUM_CORES + lax.axis_index("c")

    pltpu.sync_copy(ubc_hbm, ub_v)
    ub = [ub_v[d] for d in range(DIM)]

    def make_compute(bufA, bufB, out_v):
        def group_body(g, _):
            off = g * LANES
            acc = bufA[0, pl.ds(off, LANES)] * ub[0]
            for d in range(1, 8):
                acc = acc + bufA[d, pl.ds(off, LANES)] * ub[d]
            for d in range(8, DIM):
                acc = acc + bufB[d - 8, pl.ds(off, LANES)] * ub[d]
            out_v[pl.ds(off, LANES)] = acc
            return 0
        return lambda: lax.fori_loop(0, GROUPS, group_body, 0, unroll=False)

    compute0 = make_compute(bufA0, bufB0, out0)
    compute1 = make_compute(bufA1, bufB1, out1)

    def in_slices(c):
        col0 = c * CHUNK_COLS
        return (itT_hbm.at[pl.ds(0, 8), pl.ds(col0, CHUNK_COLS)],
                itT_hbm.at[pl.ds(8, 8), pl.ds(col0, CHUNK_COLS)])

    def start_in(c, bufA, bufB, sem):
        sa, sb = in_slices(c)
        pltpu.async_copy(sa, bufA, sem)
        pltpu.async_copy(sb, bufB, sem)

    def wait_in(c, bufA, bufB, sem):
        sa, sb = in_slices(c)
        pltpu.make_async_copy(sa, bufA, sem).wait()
        pltpu.make_async_copy(sb, bufB, sem).wait()

    def start_out(c, out_v, sem):
        pltpu.async_copy(out_v, out_hbm.at[pl.ds(c * CHUNK_COLS, CHUNK_COLS)],
                         sem)

    def wait_out(c, out_v, sem):
        pltpu.make_async_copy(
            out_v, out_hbm.at[pl.ds(c * CHUNK_COLS, CHUNK_COLS)], sem).wait()

    ncw = CHUNKS_PER_WORKER
    cid = lambda i: wid + i * NUM_WORKERS

    start_in(cid(0), bufA0, bufB0, insem0)
    npairs = (ncw + 1) // 2

    def pair_body(p, _):
        j = 2 * p

        @pl.when(j + 1 < ncw)
        def _():
            start_in(cid(j + 1), bufA1, bufB1, insem1)

        wait_in(cid(j), bufA0, bufB0, insem0)

        @pl.when(p > 0)
        def _():
            wait_out(cid(j - 2), out0, outsem0)

        compute0()
        start_out(cid(j), out0, outsem0)

        @pl.when(j + 2 < ncw)
        def _():
            start_in(cid(j + 2), bufA0, bufB0, insem0)

        @pl.when(j + 1 < ncw)
        def _():
            wait_in(cid(j + 1), bufA1, bufB1, insem1)

            @pl.when(p > 0)
            def _():
                wait_out(cid(j - 1), out1, outsem1)

            compute1()
            start_out(cid(j + 1), out1, outsem1)

        return 0

    lax.fori_loop(0, npairs, pair_body, 0, unroll=False)
    wait_out(0, out0, outsem0)
    wait_out(0, out1, outsem1)


def _tc_body(x_ref, u_ref, o_ref):
    o_ref[...] = jnp.sum(x_ref[...] * u_ref[...], axis=0)


def kernel(items_emb, user_emb):
    items_t = items_emb.T
    u_bcast_table = jnp.broadcast_to(user_emb.reshape(DIM, 1), (DIM, LANES))

    mesh = plsc.VectorSubcoreMesh(
        core_axis_name="c", subcore_axis_name="s",
        num_cores=NUM_CORES, num_subcores=NUM_SUBCORES,
    )
    sc_run = pl.kernel(
        _sc_body,
        out_type=jax.ShapeDtypeStruct((SC_ROWS,), jnp.float32),
        mesh=mesh,
        compiler_params=pltpu.CompilerParams(
            needs_layout_passes=False, use_tc_tiling_on_sc=True,
        ),
        scratch_types=[
            pltpu.VMEM((DIM, LANES), jnp.float32),
            pltpu.VMEM((8, CHUNK_COLS), jnp.float32),
            pltpu.VMEM((8, CHUNK_COLS), jnp.float32),
            pltpu.VMEM((8, CHUNK_COLS), jnp.float32),
            pltpu.VMEM((8, CHUNK_COLS), jnp.float32),
            pltpu.VMEM((CHUNK_COLS,), jnp.float32),
            pltpu.VMEM((CHUNK_COLS,), jnp.float32),
            pltpu.SemaphoreType.DMA,
            pltpu.SemaphoreType.DMA,
            pltpu.SemaphoreType.DMA,
            pltpu.SemaphoreType.DMA,
        ],
    )
    sc_out = sc_run(items_t, u_bcast_table)

    n_tc_blocks = pl.cdiv(TC_ROWS, TC_BLOCK)
    tc_out = pl.pallas_call(
        _tc_body,
        grid=(n_tc_blocks,),
        in_specs=[
            pl.BlockSpec((DIM, TC_BLOCK),
                         lambda i: (0, SC_ROWS // TC_BLOCK + i)),
            pl.BlockSpec((DIM, 1), lambda i: (0, 0)),
        ],
        out_specs=pl.BlockSpec((TC_BLOCK,), lambda i: (i,)),
        out_shape=jax.ShapeDtypeStruct((TC_ROWS,), jnp.float32),
    )(items_t, user_emb.reshape(DIM, 1))

    return jnp.concatenate([sc_out, tc_out])


# speedup vs baseline: 1.1702x; 1.1702x over previous
import functools

import jax
import jax.numpy as jnp
from jax import lax
from jax.experimental import pallas as pl
from jax.experimental.pallas import tpu as pltpu
from jax.experimental.pallas import tpu_sc as plsc

M_ROWS = 1000000
DIM = 16
LANES = 16
NUM_CORES = 2
NUM_SUBCORES = 16
NUM_WORKERS = NUM_CORES * NUM_SUBCORES

CHUNK_COLS = 1024
GROUPS = CHUNK_COLS // LANES
CHUNKS_PER_WORKER = 10
SC_ROWS = CHUNK_COLS * NUM_WORKERS * CHUNKS_PER_WORKER
TC_BLOCK = 65536
TC_BLOCK0 = SC_ROWS // TC_BLOCK


def _sc_body(itT_hbm, user_hbm, out_hbm,
             u_v, bufA0, bufB0, bufA1, bufB1, out0, out1,
             insem0, insem1, outsem0, outsem1):
    wid = lax.axis_index("s") * NUM_CORES + lax.axis_index("c")

    pltpu.sync_copy(user_hbm.at[0], u_v)
    u = u_v[...]
    ub = [
        jnp.take_along_axis(u, jnp.full((LANES,), d, jnp.int32), 0,
                            mode="promise_in_bounds")
        for d in range(DIM)
    ]

    def make_compute(bufA, bufB, out_v):
        def group_body(g, _):
            off = g * LANES
            acc = bufA[0, pl.ds(off, LANES)] * ub[0]
            for d in range(1, 8):
                acc = acc + bufA[d, pl.ds(off, LANES)] * ub[d]
            for d in range(8, DIM):
                acc = acc + bufB[d - 8, pl.ds(off, LANES)] * ub[d]
            out_v[pl.ds(off, LANES)] = acc
            return 0
        return lambda: lax.fori_loop(0, GROUPS, group_body, 0, unroll=False)

    compute0 = make_compute(bufA0, bufB0, out0)
    compute1 = make_compute(bufA1, bufB1, out1)

    def in_slices(c):
        col0 = c * CHUNK_COLS
        return (itT_hbm.at[pl.ds(0, 8), pl.ds(col0, CHUNK_COLS)],
                itT_hbm.at[pl.ds(8, 8), pl.ds(col0, CHUNK_COLS)])

    def start_in(c, bufA, bufB, sem):
        sa, sb = in_slices(c)
        pltpu.async_copy(sa, bufA, sem)
        pltpu.async_copy(sb, bufB, sem)

    def wait_in(c, bufA, bufB, sem):
        sa, sb = in_slices(c)
        pltpu.make_async_copy(sa, bufA, sem).wait()
        pltpu.make_async_copy(sb, bufB, sem).wait()

    def start_out(c, out_v, sem):
        pltpu.async_copy(out_v, out_hbm.at[pl.ds(c * CHUNK_COLS, CHUNK_COLS)],
                         sem)

    def wait_out(c, out_v, sem):
        pltpu.make_async_copy(
            out_v, out_hbm.at[pl.ds(c * CHUNK_COLS, CHUNK_COLS)], sem).wait()

    ncw = CHUNKS_PER_WORKER
    cid = lambda i: wid + i * NUM_WORKERS

    start_in(cid(0), bufA0, bufB0, insem0)
    npairs = (ncw + 1) // 2

    def pair_body(p, _):
        j = 2 * p

        @pl.when(j + 1 < ncw)
        def _():
            start_in(cid(j + 1), bufA1, bufB1, insem1)

        wait_in(cid(j), bufA0, bufB0, insem0)

        @pl.when(p > 0)
        def _():
            wait_out(cid(j - 2), out0, outsem0)

        compute0()
        start_out(cid(j), out0, outsem0)

        @pl.when(j + 2 < ncw)
        def _():
            start_in(cid(j + 2), bufA0, bufB0, insem0)

        @pl.when(j + 1 < ncw)
        def _():
            wait_in(cid(j + 1), bufA1, bufB1, insem1)

            @pl.when(p > 0)
            def _():
                wait_out(cid(j - 1), out1, outsem1)

            compute1()
            start_out(cid(j + 1), out1, outsem1)

        return 0

    lax.fori_loop(0, npairs, pair_body, 0, unroll=False)
    wait_out(0, out0, outsem0)
    wait_out(0, out1, outsem1)


def _tc_body(x_ref, u_ref, o_ref):
    prod = jax.lax.dot_general(
        u_ref[...], x_ref[...], (((1,), (0,)), ((), ())),
        preferred_element_type=jnp.float32,
    )
    o_ref[...] = prod.reshape(-1)


def kernel(items_emb, user_emb):
    items_t = items_emb.T

    mesh = plsc.VectorSubcoreMesh(
        core_axis_name="c", subcore_axis_name="s",
        num_cores=NUM_CORES, num_subcores=NUM_SUBCORES,
    )
    sc_run = pl.kernel(
        _sc_body,
        out_type=jax.ShapeDtypeStruct((SC_ROWS,), jnp.float32),
        mesh=mesh,
        compiler_params=pltpu.CompilerParams(
            needs_layout_passes=False, use_tc_tiling_on_sc=True,
        ),
        scratch_types=[
            pltpu.VMEM((LANES,), jnp.float32),
            pltpu.VMEM((8, CHUNK_COLS), jnp.float32),
            pltpu.VMEM((8, CHUNK_COLS), jnp.float32),
            pltpu.VMEM((8, CHUNK_COLS), jnp.float32),
            pltpu.VMEM((8, CHUNK_COLS), jnp.float32),
            pltpu.VMEM((CHUNK_COLS,), jnp.float32),
            pltpu.VMEM((CHUNK_COLS,), jnp.float32),
            pltpu.SemaphoreType.DMA,
            pltpu.SemaphoreType.DMA,
            pltpu.SemaphoreType.DMA,
            pltpu.SemaphoreType.DMA,
        ],
    )
    sc_out = sc_run(items_t, user_emb)

    n_tc_blocks = pl.cdiv(M_ROWS, TC_BLOCK) - TC_BLOCK0
    tc_full = pl.pallas_call(
        _tc_body,
        grid=(n_tc_blocks,),
        in_specs=[
            pl.BlockSpec((DIM, TC_BLOCK), lambda i: (0, TC_BLOCK0 + i)),
            pl.BlockSpec((1, DIM), lambda i: (0, 0)),
        ],
        out_specs=pl.BlockSpec((TC_BLOCK,), lambda i: (TC_BLOCK0 + i,)),
        out_shape=jax.ShapeDtypeStruct((M_ROWS,), jnp.float32),
    )(items_t, user_emb)

    return lax.dynamic_update_slice(tc_full, sc_out, (0,))
